# baseline (speedup 1.0000x reference)
import functools

import jax
import jax.numpy as jnp
from jax.experimental import pallas as pl
from jax.experimental.pallas import tpu as pltpu

_BF = jnp.bfloat16
_F32 = jnp.float32


def _dgt(a, b):
    return jax.lax.dot_general(a, b, (((1,), (1,)), ((), ())),
                               preferred_element_type=_F32)


def _gcn_body(ng, gp, e, x_ref, adj_ref, w1_ref, b1_ref, w2_ref, b2_ref, o_ref,
              t1_scr, h1_scr, t2_scr):
    w1 = w1_ref[...].astype(_BF)
    b1 = b1_ref[...]
    w2 = w2_ref[...]
    b2 = b2_ref[...]
    for g in range(ng):
        xg = x_ref[g].astype(_BF)
        t1_scr[g] = jnp.dot(xg, w1, preferred_element_type=_F32).astype(_BF)
    for g in range(ng):
        for i in range(gp):
            a = adj_ref[g * gp + i]
            t = jnp.dot(a, t1_scr[g], preferred_element_type=_F32)
            s = t[:, i * e:(i + 1) * e] + b1[:, i * e:(i + 1) * e]
            h1_scr[g, :, i * e:(i + 1) * e] = jnp.maximum(s, 0.0).astype(_BF)
    for g in range(ng):
        t2 = jnp.dot(h1_scr[g], w2, preferred_element_type=_F32)
        t2_scr[g] = t2.astype(_BF)
    for g in range(ng):
        for i in range(gp):
            a = adj_ref[g * gp + i]
            t = jnp.dot(a, t2_scr[g], preferred_element_type=_F32)
            s = t[:, i * e:(i + 1) * e] + b2[:, i * e:(i + 1) * e]
            h = jnp.maximum(s, 0.0).astype(_BF)
            o_ref[g * gp + i] = h.T


def _mlp_body(flat_ref, fw_ref, fb_ref, ow_ref, ob_ref, out_ref):
    f = flat_ref[...]
    z = jnp.dot(f, fw_ref[...], preferred_element_type=_F32)
    z = jnp.maximum(z + fb_ref[...], 0.0).astype(_BF)
    o = jnp.dot(z, ow_ref[...], preferred_element_type=_F32)
    out_ref[...] = o + ob_ref[...]


def kernel(x, adj, W1, b1, W2, b2, fc1_W, fc1_b, out_W, out_b):
    B, N, S = x.shape
    E = W1.shape[1]
    H = fc1_W.shape[1]
    C = out_W.shape[1]

    G = 8
    NB = min(128, B)
    MB = min(512, B)
    NG = NB // G

    eye = jnp.eye(G, dtype=_F32)
    w1bd = jnp.kron(eye, W1).astype(_BF)
    w2bd = jnp.kron(eye, W2).astype(_BF)
    b1r = jnp.tile(b1, G).reshape(1, G * E)
    b2r = jnp.tile(b2, G).reshape(1, G * E)
    fbr = fc1_b.reshape(1, H)
    obr = out_b.reshape(1, C)
    xpk = x.reshape(B // G, G, N, S).transpose(0, 2, 1, 3).reshape(B // G, N, G * S)
    fwb = fc1_W.reshape(N, E, H).transpose(1, 0, 2).reshape(N * E, H).astype(_BF)
    owb = out_W.astype(_BF)

    h2t = pl.pallas_call(
        functools.partial(_gcn_body, NG, G, E),
        grid=(B // NB,),
        in_specs=[
            pl.BlockSpec((NG, N, G * S), lambda i: (i, 0, 0)),
            pl.BlockSpec((NB, N, N), lambda i: (i, 0, 0)),
            pl.BlockSpec((G * S, G * E), lambda i: (0, 0)),
            pl.BlockSpec((1, G * E), lambda i: (0, 0)),
            pl.BlockSpec((G * E, G * E), lambda i: (0, 0)),
            pl.BlockSpec((1, G * E), lambda i: (0, 0)),
        ],
        out_specs=pl.BlockSpec((NB, E, N), lambda i: (i, 0, 0)),
        out_shape=jax.ShapeDtypeStruct((B, E, N), _BF),
        scratch_shapes=[
            pltpu.VMEM((NG, N, G * E), _BF),
            pltpu.VMEM((NG, N, G * E), _BF),
            pltpu.VMEM((NG, N, G * E), _BF),
        ],
    )(xpk, adj, w1bd, b1r, w2bd, b2r)

    flat = h2t.reshape(B, N * E)

    out = pl.pallas_call(
        _mlp_body,
        grid=(B // MB,),
        in_specs=[
            pl.BlockSpec((MB, N * E), lambda i: (i, 0)),
            pl.BlockSpec((N * E, H), lambda i: (0, 0)),
            pl.BlockSpec((1, H), lambda i: (0, 0)),
            pl.BlockSpec((H, C), lambda i: (0, 0)),
            pl.BlockSpec((1, C), lambda i: (0, 0)),
        ],
        out_specs=pl.BlockSpec((MB, C), lambda i: (i, 0)),
        out_shape=jax.ShapeDtypeStruct((B, C), _F32),
    )(flat, fwb, fbr, owb, obr)

    return out


# speedup vs baseline: 1.0980x; 1.0980x over previous
import functools

import jax
import jax.numpy as jnp
from jax.experimental import pallas as pl
from jax.experimental.pallas import tpu as pltpu

_BF = jnp.bfloat16
_F32 = jnp.float32


def _dgt(a, b):
    return jax.lax.dot_general(a, b, (((1,), (1,)), ((), ())),
                               preferred_element_type=_F32)


def _gcn_body(ng, gp, e, x_ref, adj_ref, w1_ref, b1_ref, w2_ref, b2_ref, o_ref,
              t1_scr, h1_scr, t2_scr):
    w1 = w1_ref[...].astype(_BF)
    b1 = b1_ref[...]
    w2 = w2_ref[...]
    b2 = b2_ref[...]
    for g in range(ng):
        xg = x_ref[g].astype(_BF)
        t1_scr[g] = jnp.dot(xg, w1, preferred_element_type=_F32).astype(_BF)
    for g in range(ng):
        for i in range(gp):
            a = adj_ref[g * gp + i].astype(_BF)
            t = jnp.dot(a, t1_scr[g], preferred_element_type=_F32)
            s = t[:, i * e:(i + 1) * e] + b1[:, i * e:(i + 1) * e]
            h1_scr[g, :, i * e:(i + 1) * e] = jnp.maximum(s, 0.0).astype(_BF)
    for g in range(ng):
        t2 = jnp.dot(h1_scr[g], w2, preferred_element_type=_F32)
        t2_scr[g] = t2.astype(_BF)
    for g in range(ng):
        for i in range(gp):
            a = adj_ref[g * gp + i].astype(_BF)
            t = jnp.dot(a, t2_scr[g], preferred_element_type=_F32)
            s = t[:, i * e:(i + 1) * e] + b2[:, i * e:(i + 1) * e]
            h = jnp.maximum(s, 0.0).astype(_BF)
            o_ref[g * gp + i] = h.T


def _mlp_body(flat_ref, fw_ref, fb_ref, ow_ref, ob_ref, out_ref):
    f = flat_ref[...]
    z = jnp.dot(f, fw_ref[...], preferred_element_type=_F32)
    z = jnp.maximum(z + fb_ref[...], 0.0).astype(_BF)
    o = jnp.dot(z, ow_ref[...], preferred_element_type=_F32)
    out_ref[...] = o + ob_ref[...]


def kernel(x, adj, W1, b1, W2, b2, fc1_W, fc1_b, out_W, out_b):
    B, N, S = x.shape
    E = W1.shape[1]
    H = fc1_W.shape[1]
    C = out_W.shape[1]

    G = 8
    NB = min(256, B)
    MB = min(512, B)
    NG = NB // G

    eye = jnp.eye(G, dtype=_F32)
    w1bd = jnp.kron(eye, W1).astype(_BF)
    w2bd = jnp.kron(eye, W2).astype(_BF)
    b1r = jnp.tile(b1, G).reshape(1, G * E)
    b2r = jnp.tile(b2, G).reshape(1, G * E)
    fbr = fc1_b.reshape(1, H)
    obr = out_b.reshape(1, C)
    xpk = x.reshape(B // G, G, N, S).transpose(0, 2, 1, 3).reshape(B // G, N, G * S)
    fwb = fc1_W.reshape(N, E, H).transpose(1, 0, 2).reshape(N * E, H).astype(_BF)
    owb = out_W.astype(_BF)

    h2t = pl.pallas_call(
        functools.partial(_gcn_body, NG, G, E),
        grid=(B // NB,),
        in_specs=[
            pl.BlockSpec((NG, N, G * S), lambda i: (i, 0, 0)),
            pl.BlockSpec((NB, N, N), lambda i: (i, 0, 0)),
            pl.BlockSpec((G * S, G * E), lambda i: (0, 0)),
            pl.BlockSpec((1, G * E), lambda i: (0, 0)),
            pl.BlockSpec((G * E, G * E), lambda i: (0, 0)),
            pl.BlockSpec((1, G * E), lambda i: (0, 0)),
        ],
        out_specs=pl.BlockSpec((NB, E, N), lambda i: (i, 0, 0)),
        out_shape=jax.ShapeDtypeStruct((B, E, N), _BF),
        scratch_shapes=[
            pltpu.VMEM((NG, N, G * E), _BF),
            pltpu.VMEM((NG, N, G * E), _BF),
            pltpu.VMEM((NG, N, G * E), _BF),
        ],
    )(xpk, adj, w1bd, b1r, w2bd, b2r)

    flat = h2t.reshape(B, N * E)

    out = pl.pallas_call(
        _mlp_body,
        grid=(B // MB,),
        in_specs=[
            pl.BlockSpec((MB, N * E), lambda i: (i, 0)),
            pl.BlockSpec((N * E, H), lambda i: (0, 0)),
            pl.BlockSpec((1, H), lambda i: (0, 0)),
            pl.BlockSpec((H, C), lambda i: (0, 0)),
            pl.BlockSpec((1, C), lambda i: (0, 0)),
        ],
        out_specs=pl.BlockSpec((MB, C), lambda i: (i, 0)),
        out_shape=jax.ShapeDtypeStruct((B, C), _F32),
    )(flat, fwb, fbr, owb, obr)

    return out


# speedup vs baseline: 1.1010x; 1.0027x over previous
import functools

import jax
import jax.numpy as jnp
from jax.experimental import pallas as pl
from jax.experimental.pallas import tpu as pltpu

_BF = jnp.bfloat16
_F32 = jnp.float32


def _dgt(a, b):
    return jax.lax.dot_general(a, b, (((1,), (1,)), ((), ())),
                               preferred_element_type=_F32)


def _gcn_body(ng, gp, e, x_ref, adj_ref, w1_ref, b1_ref, w2_ref, b2_ref, o_ref,
              t1_scr, h1_scr, t2_scr):
    w1 = w1_ref[...].astype(_BF)
    b1 = b1_ref[...]
    w2 = w2_ref[...]
    b2 = b2_ref[...]
    for g in range(ng):
        xg = x_ref[g].astype(_BF)
        t1_scr[g] = jnp.dot(xg, w1, preferred_element_type=_F32).astype(_BF)
    for g in range(ng):
        for i in range(gp):
            a = adj_ref[g * gp + i].astype(_BF)
            t = jnp.dot(a, t1_scr[g], preferred_element_type=_F32)
            s = t[:, i * e:(i + 1) * e] + b1[:, i * e:(i + 1) * e]
            h1_scr[g, :, i * e:(i + 1) * e] = jnp.maximum(s, 0.0).astype(_BF)
    for g in range(ng):
        t2 = jnp.dot(h1_scr[g], w2, preferred_element_type=_F32)
        t2_scr[g] = t2.astype(_BF)
    for g in range(ng):
        for i in range(gp):
            a = adj_ref[g * gp + i].astype(_BF)
            t = jnp.dot(a, t2_scr[g], preferred_element_type=_F32)
            s = t[:, i * e:(i + 1) * e] + b2[:, i * e:(i + 1) * e]
            h = jnp.maximum(s, 0.0).astype(_BF)
            o_ref[g * gp + i] = h.T


def _mlp_body(flat_ref, fw_ref, fb_ref, ow_ref, ob_ref, out_ref):
    f = flat_ref[...]
    z = jnp.dot(f, fw_ref[...], preferred_element_type=_F32)
    z = jnp.maximum(z + fb_ref[...], 0.0).astype(_BF)
    o = jnp.dot(z, ow_ref[...], preferred_element_type=_F32)
    out_ref[...] = o + ob_ref[...]


def kernel(x, adj, W1, b1, W2, b2, fc1_W, fc1_b, out_W, out_b):
    B, N, S = x.shape
    E = W1.shape[1]
    H = fc1_W.shape[1]
    C = out_W.shape[1]

    G = 8
    NB = min(256, B)
    MB = min(2048, B)
    NG = NB // G

    eye = jnp.eye(G, dtype=_F32)
    w1bd = jnp.kron(eye, W1).astype(_BF)
    w2bd = jnp.kron(eye, W2).astype(_BF)
    b1r = jnp.tile(b1, G).reshape(1, G * E)
    b2r = jnp.tile(b2, G).reshape(1, G * E)
    fbr = fc1_b.reshape(1, H)
    obr = out_b.reshape(1, C)
    xpk = x.reshape(B // G, G, N, S).transpose(0, 2, 1, 3).reshape(B // G, N, G * S)
    fwb = fc1_W.reshape(N, E, H).transpose(1, 0, 2).reshape(N * E, H).astype(_BF)
    owb = out_W.astype(_BF)

    h2t = pl.pallas_call(
        functools.partial(_gcn_body, NG, G, E),
        grid=(B // NB,),
        in_specs=[
            pl.BlockSpec((NG, N, G * S), lambda i: (i, 0, 0)),
            pl.BlockSpec((NB, N, N), lambda i: (i, 0, 0)),
            pl.BlockSpec((G * S, G * E), lambda i: (0, 0)),
            pl.BlockSpec((1, G * E), lambda i: (0, 0)),
            pl.BlockSpec((G * E, G * E), lambda i: (0, 0)),
            pl.BlockSpec((1, G * E), lambda i: (0, 0)),
        ],
        out_specs=pl.BlockSpec((NB, E, N), lambda i: (i, 0, 0)),
        out_shape=jax.ShapeDtypeStruct((B, E, N), _BF),
        scratch_shapes=[
            pltpu.VMEM((NG, N, G * E), _BF),
            pltpu.VMEM((NG, N, G * E), _BF),
            pltpu.VMEM((NG, N, G * E), _BF),
        ],
    )(xpk, adj, w1bd, b1r, w2bd, b2r)

    flat = h2t.reshape(B, N * E)

    out = pl.pallas_call(
        _mlp_body,
        grid=(B // MB,),
        in_specs=[
            pl.BlockSpec((MB, N * E), lambda i: (i, 0)),
            pl.BlockSpec((N * E, H), lambda i: (0, 0)),
            pl.BlockSpec((1, H), lambda i: (0, 0)),
            pl.BlockSpec((H, C), lambda i: (0, 0)),
            pl.BlockSpec((1, C), lambda i: (0, 0)),
        ],
        out_specs=pl.BlockSpec((MB, C), lambda i: (i, 0)),
        out_shape=jax.ShapeDtypeStruct((B, C), _F32),
    )(flat, fwb, fbr, owb, obr)

    return out
